# baseline (speedup 1.0000x reference)
import jax
import jax.numpy as jnp
from jax import lax
from jax.experimental import pallas as pl
from jax.experimental.pallas import tpu as pltpu
from jax.experimental.pallas import tpu_sc as plsc

EPS = 1e-08

NC = 2
NS = 16
NW = NC * NS
CH = 128
MCH = 80
EP = NW * CH * MCH
PP = 16


def _silu(x):
    return x * jax.nn.sigmoid(x)


def _prep_body(h_ref, a_ref, b_ref, hr_ref, hc_ref):
    h = h_ref[...]
    hr_ref[...] = jnp.dot(h, a_ref[...],
                          preferred_element_type=jnp.float32
                          ).astype(jnp.bfloat16)
    hc_ref[...] = jnp.dot(h, b_ref[...],
                          preferred_element_type=jnp.float32
                          ).astype(jnp.bfloat16)


def _prep(h, wa, wb, bn):
    n, d = h.shape
    grid = n // bn
    return pl.pallas_call(
        _prep_body,
        grid=(grid,),
        in_specs=[
            pl.BlockSpec((bn, d), lambda i: (i, 0)),
            pl.BlockSpec((d, d), lambda i: (0, 0)),
            pl.BlockSpec((d, d), lambda i: (0, 0)),
        ],
        out_specs=[
            pl.BlockSpec((bn, d), lambda i: (i, 0)),
            pl.BlockSpec((bn, d), lambda i: (i, 0)),
        ],
        out_shape=[
            jax.ShapeDtypeStruct((n, d), jnp.bfloat16),
            jax.ShapeDtypeStruct((n, d), jnp.bfloat16),
        ],
        compiler_params=pltpu.CompilerParams(
            dimension_semantics=("parallel",)),
    )(h, wa, wb)


def _gather_body(row_hbm, col_hbm, hr_hbm, hc_hbm, pp_hbm,
                 g1_hbm, g2_hbm, dif_hbm,
                 idx_r, idx_c, buf_a, buf_b, pa, pb, sem0, sem1, wsem0, wsem1):
    wid = lax.axis_index("s") * NC + lax.axis_index("c")
    w_base = wid * MCH * CH
    sems = (sem0, sem1)
    wsems = (wsem0, wsem1)

    def fire(c, b):
        base = w_base + c * CH
        pltpu.sync_copy(row_hbm.at[pl.ds(base, CH)], idx_r.at[b])
        pltpu.sync_copy(col_hbm.at[pl.ds(base, CH)], idx_c.at[b])
        pltpu.async_copy(hr_hbm.at[idx_r.at[b]], buf_a.at[b], sems[b])
        pltpu.async_copy(hc_hbm.at[idx_c.at[b]], buf_b.at[b], sems[b])
        pltpu.async_copy(pp_hbm.at[idx_r.at[b]], pa.at[b], sems[b])
        pltpu.async_copy(pp_hbm.at[idx_c.at[b]], pb.at[b], sems[b])

    def wait(b):
        pltpu.make_async_copy(hr_hbm.at[idx_r.at[b]], buf_a.at[b],
                              sems[b]).wait()
        pltpu.make_async_copy(hc_hbm.at[idx_c.at[b]], buf_b.at[b],
                              sems[b]).wait()
        pltpu.make_async_copy(pp_hbm.at[idx_r.at[b]], pa.at[b],
                              sems[b]).wait()
        pltpu.make_async_copy(pp_hbm.at[idx_c.at[b]], pb.at[b],
                              sems[b]).wait()

    def fire_writes(c, b):
        base = w_base + c * CH
        pltpu.async_copy(buf_a.at[b], g1_hbm.at[pl.ds(base, CH)], wsems[b])
        pltpu.async_copy(buf_b.at[b], g2_hbm.at[pl.ds(base, CH)], wsems[b])
        pltpu.async_copy(pa.at[b], dif_hbm.at[pl.ds(base, CH)], wsems[b])

    def wait_writes(c, b):
        base = w_base + c * CH
        pltpu.make_async_copy(buf_a.at[b], g1_hbm.at[pl.ds(base, CH)],
                              wsems[b]).wait()
        pltpu.make_async_copy(buf_b.at[b], g2_hbm.at[pl.ds(base, CH)],
                              wsems[b]).wait()
        pltpu.make_async_copy(pa.at[b], dif_hbm.at[pl.ds(base, CH)],
                              wsems[b]).wait()

    fire(0, 0)

    @pl.loop(0, MCH // 2)
    def _pair(i):
        for b in range(2):
            j = i * 2 + b

            @pl.when(j < MCH - 1)
            def _():
                @pl.when(j >= 1)
                def _():
                    wait_writes(j - 1, 1 - b)

                fire(j + 1, 1 - b)

            wait(b)

            @pl.loop(0, CH)
            def _row(r):
                pa[b, r, :] = pa[b, r, :] - pb[b, r, :]

            fire_writes(j, b)

    wait_writes(MCH - 2, 0)
    wait_writes(MCH - 1, 1)


def _gather(row, col, hr, hc, pos_pad):
    d = hr.shape[1]
    mesh = plsc.VectorSubcoreMesh(core_axis_name="c", subcore_axis_name="s",
                                  num_cores=NC, num_subcores=NS)
    f = pl.kernel(
        _gather_body,
        out_type=[
            jax.ShapeDtypeStruct((EP, d), jnp.bfloat16),
            jax.ShapeDtypeStruct((EP, d), jnp.bfloat16),
            jax.ShapeDtypeStruct((EP, PP), jnp.float32),
        ],
        mesh=mesh,
        scratch_types=[
            pltpu.VMEM((2, CH), jnp.int32),
            pltpu.VMEM((2, CH), jnp.int32),
            pltpu.VMEM((2, CH, d), jnp.bfloat16),
            pltpu.VMEM((2, CH, d), jnp.bfloat16),
            pltpu.VMEM((2, CH, PP), jnp.float32),
            pltpu.VMEM((2, CH, PP), jnp.float32),
            pltpu.SemaphoreType.DMA,
            pltpu.SemaphoreType.DMA,
            pltpu.SemaphoreType.DMA,
            pltpu.SemaphoreType.DMA,
        ],
        compiler_params=pltpu.CompilerParams(use_tc_tiling_on_sc=False),
    )
    return f(row, col, hr, hc, pos_pad)


def _edge_body(ne, be, g1_ref, g2_ref, eat_ref, dif_ref,
               wea_ref, wrad_ref, be1_ref, we2_ref, be2_ref,
               wc1_ref, bc1_ref, wc2_ref,
               e_ref, t_ref):
    bf = jnp.bfloat16
    diff = dif_ref[...]
    radial = jnp.sqrt(jnp.sum(diff * diff, axis=1, keepdims=True)) + EPS
    pre = (g1_ref[...].astype(jnp.float32) + g2_ref[...].astype(jnp.float32)
           + lax.dot_general(eat_ref[...].astype(bf),
                             wea_ref[...].astype(bf),
                             dimension_numbers=(((0,), (0,)), ((), ())),
                             preferred_element_type=jnp.float32)
           + radial * wrad_ref[...]
           + be1_ref[...])
    e1 = _silu(pre)
    e2 = _silu(jnp.dot(e1.astype(bf), we2_ref[...].astype(bf),
                       preferred_element_type=jnp.float32) + be2_ref[...])
    c1 = _silu(jnp.dot(e2.astype(bf), wc1_ref[...].astype(bf),
                       preferred_element_type=jnp.float32) + bc1_ref[...])
    cu = jnp.sum(c1 * wc2_ref[...], axis=1, keepdims=True)
    cu = jnp.clip(cu, -1.0, 1.0)
    erow = pl.program_id(0) * be + lax.broadcasted_iota(jnp.int32, (be, 1), 0)
    valid = erow < ne
    e_ref[...] = jnp.where(valid, e2, 0.0)
    t_ref[...] = jnp.where(valid, cu * diff, 0.0)


def _edge_mlp(g1, g2, eat, dif, wea, wrad, be1, we2, be2, wc1, bc1, wc2,
              ne, be):
    ep, d = g1.shape
    ed = eat.shape[0]
    grid = ep // be
    ea_max = ne // be - 1
    full = lambda i: (0, 0)
    import functools
    return pl.pallas_call(
        functools.partial(_edge_body, ne, be),
        grid=(grid,),
        in_specs=[
            pl.BlockSpec((be, d), lambda i: (i, 0)),
            pl.BlockSpec((be, d), lambda i: (i, 0)),
            pl.BlockSpec((ed, be), lambda i, m=ea_max: (0, jnp.minimum(i, m))),
            pl.BlockSpec((be, PP), lambda i: (i, 0)),
            pl.BlockSpec((ed, d), full),
            pl.BlockSpec((1, d), full),
            pl.BlockSpec((1, d), full),
            pl.BlockSpec((d, d), full),
            pl.BlockSpec((1, d), full),
            pl.BlockSpec((d, d), full),
            pl.BlockSpec((1, d), full),
            pl.BlockSpec((1, d), full),
        ],
        out_specs=[
            pl.BlockSpec((be, d), lambda i: (i, 0)),
            pl.BlockSpec((be, PP), lambda i: (i, 0)),
        ],
        out_shape=[
            jax.ShapeDtypeStruct((ep, d), jnp.float32),
            jax.ShapeDtypeStruct((ep, PP), jnp.float32),
        ],
        compiler_params=pltpu.CompilerParams(
            dimension_semantics=("parallel",)),
    )(g1, g2, eat, dif, wea, wrad, be1, we2, be2, wc1, bc1, wc2)


def _scatter_body(row_hbm, e_hbm, t_hbm, zn_hbm, zc_hbm,
                  outn_hbm, outc_hbm,
                  idx, ebuf, tbuf, accn, accc, sem0, sem1):
    n = zn_hbm.shape[0]
    rows_per_s = n // NS
    c = lax.axis_index("c")
    s = lax.axis_index("s")
    wid = s * NC + c
    w_base = wid * MCH * CH
    sems = (sem0, sem1)

    pltpu.sync_copy(zn_hbm.at[pl.ds(s * rows_per_s, rows_per_s)],
                    accn.at[pl.ds(s * rows_per_s, rows_per_s)])
    pltpu.sync_copy(zc_hbm.at[pl.ds(s * rows_per_s, rows_per_s)],
                    accc.at[pl.ds(s * rows_per_s, rows_per_s)])
    plsc.subcore_barrier()

    def fire(j, b):
        base = w_base + j * CH
        pltpu.async_copy(row_hbm.at[pl.ds(base, CH)], idx.at[b], sems[b])
        pltpu.async_copy(e_hbm.at[pl.ds(base, CH)], ebuf.at[b], sems[b])
        pltpu.async_copy(t_hbm.at[pl.ds(base, CH)], tbuf.at[b], sems[b])

    def wait(j, b):
        base = w_base + j * CH
        pltpu.make_async_copy(row_hbm.at[pl.ds(base, CH)], idx.at[b],
                              sems[b]).wait()
        pltpu.make_async_copy(e_hbm.at[pl.ds(base, CH)], ebuf.at[b],
                              sems[b]).wait()
        pltpu.make_async_copy(t_hbm.at[pl.ds(base, CH)], tbuf.at[b],
                              sems[b]).wait()

    fire(0, 0)

    @pl.loop(0, MCH // 2)
    def _pair(i):
        for b in range(2):
            j = i * 2 + b

            @pl.when(j < MCH - 1)
            def _():
                fire(j + 1, 1 - b)

            wait(j, b)
            pltpu.sync_copy(ebuf.at[b], accn.at[idx.at[b]], add=True)
            pltpu.sync_copy(tbuf.at[b], accc.at[idx.at[b]], add=True)

    plsc.subcore_barrier()
    pltpu.sync_copy(accn.at[pl.ds(s * rows_per_s, rows_per_s)],
                    outn_hbm.at[pl.ds(c * n + s * rows_per_s, rows_per_s)])
    pltpu.sync_copy(accc.at[pl.ds(s * rows_per_s, rows_per_s)],
                    outc_hbm.at[pl.ds(c * n + s * rows_per_s, rows_per_s)])


def _scatter(row, earr, tarr, n):
    ep, d = earr.shape
    zn = jnp.zeros((n, d), jnp.float32)
    zc = jnp.zeros((n, PP), jnp.float32)
    mesh = plsc.VectorSubcoreMesh(core_axis_name="c", subcore_axis_name="s",
                                  num_cores=NC, num_subcores=NS)
    f = pl.kernel(
        _scatter_body,
        out_type=[
            jax.ShapeDtypeStruct((NC * n, d), jnp.float32),
            jax.ShapeDtypeStruct((NC * n, PP), jnp.float32),
        ],
        mesh=mesh,
        scratch_types=[
            pltpu.VMEM((2, CH), jnp.int32),
            pltpu.VMEM((2, CH, d), jnp.float32),
            pltpu.VMEM((2, CH, PP), jnp.float32),
            pltpu.VMEM_SHARED((n, d), jnp.float32),
            pltpu.VMEM_SHARED((n, PP), jnp.float32),
            pltpu.SemaphoreType.DMA,
            pltpu.SemaphoreType.DMA,
        ],
        compiler_params=pltpu.CompilerParams(use_tc_tiling_on_sc=False),
    )
    return f(row, earr, tarr, zn, zc)


def _node_body(h_ref, n1_ref, n2_ref, c1_ref, c2_ref, pp_ref,
               wn1a_ref, wn1b_ref, bn1_ref, wn2_ref, bn2_ref,
               hn_ref, pn_ref):
    h = h_ref[...]
    an = n1_ref[...] + n2_ref[...]
    x = _silu(jnp.dot(h, wn1a_ref[...], preferred_element_type=jnp.float32)
              + jnp.dot(an, wn1b_ref[...], preferred_element_type=jnp.float32)
              + bn1_ref[...])
    hn_ref[...] = (jnp.dot(x, wn2_ref[...], preferred_element_type=jnp.float32)
                   + bn2_ref[...] + h)
    pn_ref[...] = pp_ref[...] + c1_ref[...] + c2_ref[...]


def _node_mlp(h, outn, outc, pos_pad, wn1a, wn1b, bn1, wn2, bn2, bn):
    n, d = h.shape
    grid = n // bn
    full = lambda i: (0, 0)
    return pl.pallas_call(
        _node_body,
        grid=(grid,),
        in_specs=[
            pl.BlockSpec((bn, d), lambda i: (i, 0)),
            pl.BlockSpec((bn, d), lambda i: (i, 0)),
            pl.BlockSpec((bn, d), lambda i, g=grid: (i + g, 0)),
            pl.BlockSpec((bn, PP), lambda i: (i, 0)),
            pl.BlockSpec((bn, PP), lambda i, g=grid: (i + g, 0)),
            pl.BlockSpec((bn, PP), lambda i: (i, 0)),
            pl.BlockSpec((d, d), full),
            pl.BlockSpec((d, d), full),
            pl.BlockSpec((1, d), full),
            pl.BlockSpec((d, d), full),
            pl.BlockSpec((1, d), full),
        ],
        out_specs=[
            pl.BlockSpec((bn, d), lambda i: (i, 0)),
            pl.BlockSpec((bn, PP), lambda i: (i, 0)),
        ],
        out_shape=[
            jax.ShapeDtypeStruct((n, d), jnp.float32),
            jax.ShapeDtypeStruct((n, PP), jnp.float32),
        ],
        compiler_params=pltpu.CompilerParams(
            dimension_semantics=("parallel",)),
    )(h, outn, outn, outc, outc, pos_pad, wn1a, wn1b, bn1, wn2, bn2)


def kernel(h, edge_index, edge_attr, pos, We1, be1, We2, be2,
           Wc1, bc1, Wc2, Wn1, bn1, Wn2, bn2):
    n, d = h.shape
    e = edge_index.shape[1]
    ed = edge_attr.shape[1]

    row = jnp.zeros((EP,), jnp.int32).at[:e].set(edge_index[0])
    col = jnp.zeros((EP,), jnp.int32).at[:e].set(edge_index[1])
    pos_pad = jnp.zeros((n, PP), jnp.float32).at[:, :3].set(pos)

    wa = We1[:d]
    wb = We1[d:2 * d]
    wea = We1[2 * d:2 * d + ed]
    wrad = We1[2 * d + ed:]

    hr, hc = _prep(h, wa, wb, bn=2000)

    g1, g2, dif = _gather(row, col, hr, hc, pos_pad)

    earr, tarr = _edge_mlp(g1, g2, edge_attr.T, dif,
                           wea, wrad, be1.reshape(1, -1), We2,
                           be2.reshape(1, -1), Wc1, bc1.reshape(1, -1),
                           Wc2.reshape(1, -1), ne=e, be=2560)

    outn, outc = _scatter(row, earr, tarr, n)

    h_new, pn = _node_mlp(h, outn, outc, pos_pad,
                          Wn1[:d], Wn1[d:], bn1.reshape(1, -1),
                          Wn2, bn2.reshape(1, -1), bn=2000)

    return (h_new, pn[:, :3])


# speedup vs baseline: 1.4092x; 1.4092x over previous
import numpy as np
import jax
import jax.numpy as jnp
from jax import lax
from jax.experimental import pallas as pl
from jax.experimental.pallas import tpu as pltpu
from jax.experimental.pallas import tpu_sc as plsc

EPS = 1e-08

NC = 2
NS = 16
NW = NC * NS
CH = 128
MCH = 80
EP = NW * CH * MCH
PP = 16


def _silu(x):
    return x * jax.nn.sigmoid(x)


_PERM = np.zeros(128, np.int32)
for _c in range(4):
    for _i in range(16):
        _PERM[_c * 32 + 2 * _i] = _c * 32 + _i
        _PERM[_c * 32 + 2 * _i + 1] = _c * 32 + 16 + _i


def _prep_body(h_ref, a_ref, b_ref, pb_ref, hr_ref, hc_ref):
    h = h_ref[...]
    pb = pb_ref[...]
    mr = jnp.dot(h, a_ref[...],
                 preferred_element_type=jnp.float32).astype(jnp.bfloat16)
    mc = jnp.dot(h, b_ref[...],
                 preferred_element_type=jnp.float32).astype(jnp.bfloat16)
    hr_ref[...] = jnp.concatenate([mr, pb], axis=1)
    hc_ref[...] = jnp.concatenate([mc, pb], axis=1)


def _prep(h, wa, wb, posbf, bn):
    n, d = h.shape
    dt = d + posbf.shape[1]
    grid = n // bn
    return pl.pallas_call(
        _prep_body,
        grid=(grid,),
        in_specs=[
            pl.BlockSpec((bn, d), lambda i: (i, 0)),
            pl.BlockSpec((d, d), lambda i: (0, 0)),
            pl.BlockSpec((d, d), lambda i: (0, 0)),
            pl.BlockSpec((bn, posbf.shape[1]), lambda i: (i, 0)),
        ],
        out_specs=[
            pl.BlockSpec((bn, dt), lambda i: (i, 0)),
            pl.BlockSpec((bn, dt), lambda i: (i, 0)),
        ],
        out_shape=[
            jax.ShapeDtypeStruct((n, dt), jnp.bfloat16),
            jax.ShapeDtypeStruct((n, dt), jnp.bfloat16),
        ],
        compiler_params=pltpu.CompilerParams(
            dimension_semantics=("parallel",)),
    )(h, wa, wb, posbf)


def _gather_body(rc_hbm, hr_hbm, hc_hbm,
                 g_hbm, dif_hbm,
                 idx, buf_a, buf_b, gb, db, sem0, sem1, wsem0, wsem1):
    d = 128
    wid = lax.axis_index("s") * NC + lax.axis_index("c")
    w_base = wid * MCH * CH
    sems = (sem0, sem1)
    wsems = (wsem0, wsem1)
    comp = plsc.PackFormat.INTERLEAVED

    def fire(c, b):
        base = w_base + c * CH
        pltpu.sync_copy(rc_hbm.at[pl.ds(base * 2, 2 * CH)], idx.at[b])
        pltpu.async_copy(hr_hbm.at[idx.at[b, pl.ds(0, CH)]], buf_a.at[b],
                         sems[b])
        pltpu.async_copy(hc_hbm.at[idx.at[b, pl.ds(CH, CH)]], buf_b.at[b],
                         sems[b])

    def wait(b):
        pltpu.make_async_copy(hr_hbm.at[idx.at[b, pl.ds(0, CH)]],
                              buf_a.at[b], sems[b]).wait()
        pltpu.make_async_copy(hc_hbm.at[idx.at[b, pl.ds(CH, CH)]],
                              buf_b.at[b], sems[b]).wait()

    def fire_writes(c, b):
        base = w_base + c * CH
        pltpu.async_copy(gb.at[b], g_hbm.at[pl.ds(base, CH)], wsems[b])
        pltpu.async_copy(db.at[b], dif_hbm.at[pl.ds(base, CH)], wsems[b])

    def wait_writes(c, b):
        base = w_base + c * CH
        pltpu.make_async_copy(gb.at[b], g_hbm.at[pl.ds(base, CH)],
                              wsems[b]).wait()
        pltpu.make_async_copy(db.at[b], dif_hbm.at[pl.ds(base, CH)],
                              wsems[b]).wait()

    fire(0, 0)

    @pl.loop(0, MCH // 2)
    def _pair(i):
        for b in range(2):
            j = i * 2 + b

            @pl.when(j < MCH - 1)
            def _():
                @pl.when(j >= 1)
                def _():
                    wait_writes(j - 1, 1 - b)

                fire(j + 1, 1 - b)

            wait(b)

            @pl.loop(0, CH, unroll=2)
            def _row(r):
                for c in range(d // 32):
                    sl = pl.ds(c * 32, 32)
                    a0, a1 = plsc.unpack(
                        buf_a[b, r, sl], format=comp,
                        preferred_element_type=jnp.float32)
                    b0, b1 = plsc.unpack(
                        buf_b[b, r, sl], format=comp,
                        preferred_element_type=jnp.float32)
                    gb[b, r, pl.ds(c * 32, 16)] = a0 + b0
                    gb[b, r, pl.ds(c * 32 + 16, 16)] = a1 + b1
                pr = plsc.bitcast(buf_a[b, r, pl.ds(d, 32)], jnp.float32)
                pc = plsc.bitcast(buf_b[b, r, pl.ds(d, 32)], jnp.float32)
                db[b, r, :] = pr - pc

            fire_writes(j, b)

    wait_writes(MCH - 2, 0)
    wait_writes(MCH - 1, 1)


def _gather(rc, hrp, hcp):
    d = 128
    dt = hrp.shape[1]
    mesh = plsc.VectorSubcoreMesh(core_axis_name="c", subcore_axis_name="s",
                                  num_cores=NC, num_subcores=NS)
    f = pl.kernel(
        _gather_body,
        out_type=[
            jax.ShapeDtypeStruct((EP, d), jnp.float32),
            jax.ShapeDtypeStruct((EP, PP), jnp.float32),
        ],
        mesh=mesh,
        scratch_types=[
            pltpu.VMEM((2, 2 * CH), jnp.int32),
            pltpu.VMEM((2, CH, dt), jnp.bfloat16),
            pltpu.VMEM((2, CH, dt), jnp.bfloat16),
            pltpu.VMEM((2, CH, d), jnp.float32),
            pltpu.VMEM((2, CH, PP), jnp.float32),
            pltpu.SemaphoreType.DMA,
            pltpu.SemaphoreType.DMA,
            pltpu.SemaphoreType.DMA,
            pltpu.SemaphoreType.DMA,
        ],
        compiler_params=pltpu.CompilerParams(use_tc_tiling_on_sc=False,
                                             needs_layout_passes=False),
    )
    return f(rc, hrp, hcp)


def _edge_body(ne, be, g_ref, eat_ref, dif_ref,
               wea_ref, wrad_ref, be1_ref, we2_ref, be2_ref,
               wc1_ref, bc1_ref, wc2_ref,
               e_ref, t_ref):
    bf = jnp.bfloat16
    diff = dif_ref[...]
    radial = jnp.sqrt(jnp.sum(diff * diff, axis=1, keepdims=True)) + EPS
    pre = (g_ref[...]
           + lax.dot_general(eat_ref[...].astype(bf),
                             wea_ref[...].astype(bf),
                             dimension_numbers=(((0,), (0,)), ((), ())),
                             preferred_element_type=jnp.float32)
           + radial * wrad_ref[...]
           + be1_ref[...])
    e1 = _silu(pre)
    e2 = _silu(jnp.dot(e1.astype(bf), we2_ref[...].astype(bf),
                       preferred_element_type=jnp.float32) + be2_ref[...])
    c1 = _silu(jnp.dot(e2.astype(bf), wc1_ref[...].astype(bf),
                       preferred_element_type=jnp.float32) + bc1_ref[...])
    cu = jnp.sum(c1 * wc2_ref[...], axis=1, keepdims=True)
    cu = jnp.clip(cu, -1.0, 1.0)
    erow = pl.program_id(0) * be + lax.broadcasted_iota(jnp.int32, (be, 1), 0)
    valid = erow < ne
    e_ref[...] = jnp.where(valid, e2, 0.0)
    t_ref[...] = jnp.where(valid, cu * diff, 0.0)


def _edge_mlp(g, eat, dif, wea, wrad, be1, we2, be2, wc1, bc1, wc2,
              ne, be):
    ep, d = g.shape
    ed = eat.shape[0]
    grid = ep // be
    ea_max = ne // be - 1
    full = lambda i: (0, 0)
    import functools
    return pl.pallas_call(
        functools.partial(_edge_body, ne, be),
        grid=(grid,),
        in_specs=[
            pl.BlockSpec((be, d), lambda i: (i, 0)),
            pl.BlockSpec((ed, be), lambda i, m=ea_max: (0, jnp.minimum(i, m))),
            pl.BlockSpec((be, PP), lambda i: (i, 0)),
            pl.BlockSpec((ed, d), full),
            pl.BlockSpec((1, d), full),
            pl.BlockSpec((1, d), full),
            pl.BlockSpec((d, d), full),
            pl.BlockSpec((1, d), full),
            pl.BlockSpec((d, d), full),
            pl.BlockSpec((1, d), full),
            pl.BlockSpec((1, d), full),
        ],
        out_specs=[
            pl.BlockSpec((be, d), lambda i: (i, 0)),
            pl.BlockSpec((be, PP), lambda i: (i, 0)),
        ],
        out_shape=[
            jax.ShapeDtypeStruct((ep, d), jnp.float32),
            jax.ShapeDtypeStruct((ep, PP), jnp.float32),
        ],
        compiler_params=pltpu.CompilerParams(
            dimension_semantics=("parallel",)),
    )(g, eat, dif, wea, wrad, be1, we2, be2, wc1, bc1, wc2)


def _scatter_body(row_hbm, e_hbm, t_hbm, zn_hbm, zc_hbm,
                  outn_hbm, outc_hbm,
                  idx, ebuf, tbuf, accn, accc, sem0, sem1):
    n = zn_hbm.shape[0]
    rows_per_s = n // NS
    c = lax.axis_index("c")
    s = lax.axis_index("s")
    wid = s * NC + c
    w_base = wid * MCH * CH
    sems = (sem0, sem1)

    pltpu.sync_copy(zn_hbm.at[pl.ds(s * rows_per_s, rows_per_s)],
                    accn.at[pl.ds(s * rows_per_s, rows_per_s)])
    pltpu.sync_copy(zc_hbm.at[pl.ds(s * rows_per_s, rows_per_s)],
                    accc.at[pl.ds(s * rows_per_s, rows_per_s)])
    plsc.subcore_barrier()

    def fire(j, b):
        base = w_base + j * CH
        pltpu.async_copy(row_hbm.at[pl.ds(base, CH)], idx.at[b], sems[b])
        pltpu.async_copy(e_hbm.at[pl.ds(base, CH)], ebuf.at[b], sems[b])
        pltpu.async_copy(t_hbm.at[pl.ds(base, CH)], tbuf.at[b], sems[b])

    def wait(j, b):
        base = w_base + j * CH
        pltpu.make_async_copy(row_hbm.at[pl.ds(base, CH)], idx.at[b],
                              sems[b]).wait()
        pltpu.make_async_copy(e_hbm.at[pl.ds(base, CH)], ebuf.at[b],
                              sems[b]).wait()
        pltpu.make_async_copy(t_hbm.at[pl.ds(base, CH)], tbuf.at[b],
                              sems[b]).wait()

    fire(0, 0)

    @pl.loop(0, MCH // 2)
    def _pair(i):
        for b in range(2):
            j = i * 2 + b

            @pl.when(j < MCH - 1)
            def _():
                fire(j + 1, 1 - b)

            wait(j, b)
            pltpu.sync_copy(ebuf.at[b], accn.at[idx.at[b]], add=True)
            pltpu.sync_copy(tbuf.at[b], accc.at[idx.at[b]], add=True)

    plsc.subcore_barrier()
    pltpu.sync_copy(accn.at[pl.ds(s * rows_per_s, rows_per_s)],
                    outn_hbm.at[pl.ds(c * n + s * rows_per_s, rows_per_s)])
    pltpu.sync_copy(accc.at[pl.ds(s * rows_per_s, rows_per_s)],
                    outc_hbm.at[pl.ds(c * n + s * rows_per_s, rows_per_s)])


def _scatter(row, earr, tarr, n):
    ep, d = earr.shape
    zn = jnp.zeros((n, d), jnp.float32)
    zc = jnp.zeros((n, PP), jnp.float32)
    mesh = plsc.VectorSubcoreMesh(core_axis_name="c", subcore_axis_name="s",
                                  num_cores=NC, num_subcores=NS)
    f = pl.kernel(
        _scatter_body,
        out_type=[
            jax.ShapeDtypeStruct((NC * n, d), jnp.float32),
            jax.ShapeDtypeStruct((NC * n, PP), jnp.float32),
        ],
        mesh=mesh,
        scratch_types=[
            pltpu.VMEM((2, CH), jnp.int32),
            pltpu.VMEM((2, CH, d), jnp.float32),
            pltpu.VMEM((2, CH, PP), jnp.float32),
            pltpu.VMEM_SHARED((n, d), jnp.float32),
            pltpu.VMEM_SHARED((n, PP), jnp.float32),
            pltpu.SemaphoreType.DMA,
            pltpu.SemaphoreType.DMA,
        ],
        compiler_params=pltpu.CompilerParams(use_tc_tiling_on_sc=False),
    )
    return f(row, earr, tarr, zn, zc)


def _node_body(h_ref, n1_ref, n2_ref, c1_ref, c2_ref, pp_ref,
               wn1a_ref, wn1b_ref, bn1_ref, wn2_ref, bn2_ref,
               hn_ref, pn_ref):
    h = h_ref[...]
    an = n1_ref[...] + n2_ref[...]
    x = _silu(jnp.dot(h, wn1a_ref[...], preferred_element_type=jnp.float32)
              + jnp.dot(an, wn1b_ref[...], preferred_element_type=jnp.float32)
              + bn1_ref[...])
    hn_ref[...] = (jnp.dot(x, wn2_ref[...], preferred_element_type=jnp.float32)
                   + bn2_ref[...] + h)
    pn_ref[...] = pp_ref[...] + c1_ref[...] + c2_ref[...]


def _node_mlp(h, outn, outc, pos_pad, wn1a, wn1b, bn1, wn2, bn2, bn):
    n, d = h.shape
    grid = n // bn
    full = lambda i: (0, 0)
    return pl.pallas_call(
        _node_body,
        grid=(grid,),
        in_specs=[
            pl.BlockSpec((bn, d), lambda i: (i, 0)),
            pl.BlockSpec((bn, d), lambda i: (i, 0)),
            pl.BlockSpec((bn, d), lambda i, g=grid: (i + g, 0)),
            pl.BlockSpec((bn, PP), lambda i: (i, 0)),
            pl.BlockSpec((bn, PP), lambda i, g=grid: (i + g, 0)),
            pl.BlockSpec((bn, PP), lambda i: (i, 0)),
            pl.BlockSpec((d, d), full),
            pl.BlockSpec((d, d), full),
            pl.BlockSpec((1, d), full),
            pl.BlockSpec((d, d), full),
            pl.BlockSpec((1, d), full),
        ],
        out_specs=[
            pl.BlockSpec((bn, d), lambda i: (i, 0)),
            pl.BlockSpec((bn, PP), lambda i: (i, 0)),
        ],
        out_shape=[
            jax.ShapeDtypeStruct((n, d), jnp.float32),
            jax.ShapeDtypeStruct((n, PP), jnp.float32),
        ],
        compiler_params=pltpu.CompilerParams(
            dimension_semantics=("parallel",)),
    )(h, outn, outn, outc, outc, pos_pad, wn1a, wn1b, bn1, wn2, bn2)


def kernel(h, edge_index, edge_attr, pos, We1, be1, We2, be2,
           Wc1, bc1, Wc2, Wn1, bn1, Wn2, bn2):
    n, d = h.shape
    e = edge_index.shape[1]
    ed = edge_attr.shape[1]

    row = jnp.zeros((EP,), jnp.int32).at[:e].set(edge_index[0])
    col = jnp.zeros((EP,), jnp.int32).at[:e].set(edge_index[1])
    pos_pad = jnp.zeros((n, PP), jnp.float32).at[:, :3].set(pos)
    posbf = lax.bitcast_convert_type(pos_pad, jnp.bfloat16).reshape(n, 2 * PP)
    rc = jnp.stack([row.reshape(-1, CH), col.reshape(-1, CH)],
                   axis=1).reshape(-1)

    wa = We1[:d][:, _PERM]
    wb = We1[d:2 * d][:, _PERM]
    wea = We1[2 * d:2 * d + ed]
    wrad = We1[2 * d + ed:]

    hrp, hcp = _prep(h, wa, wb, posbf, bn=2000)

    g, dif = _gather(rc, hrp, hcp)

    earr, tarr = _edge_mlp(g, edge_attr.T, dif,
                           wea, wrad, be1.reshape(1, -1), We2,
                           be2.reshape(1, -1), Wc1, bc1.reshape(1, -1),
                           Wc2.reshape(1, -1), ne=e, be=2560)

    outn, outc = _scatter(row, earr, tarr, n)

    h_new, pn = _node_mlp(h, outn, outc, pos_pad,
                          Wn1[:d], Wn1[d:], bn1.reshape(1, -1),
                          Wn2, bn2.reshape(1, -1), bn=2000)

    return (h_new, pn[:, :3])


# speedup vs baseline: 1.5045x; 1.0677x over previous
import functools

import numpy as np
import jax
import jax.numpy as jnp
from jax import lax
from jax.experimental import pallas as pl
from jax.experimental.pallas import tpu as pltpu
from jax.experimental.pallas import tpu_sc as plsc

EPS = 1e-08

NC = 2
NS = 16
NW = NC * NS
CH = 128
MCH = 80
EP = NW * CH * MCH
PP = 16


def _silu(x):
    return x * jax.nn.sigmoid(x)


_PERM = np.zeros(128, np.int32)
for _c in range(4):
    for _i in range(16):
        _PERM[_c * 32 + 2 * _i] = _c * 32 + _i
        _PERM[_c * 32 + 2 * _i + 1] = _c * 32 + 16 + _i


def _prep_body(h_ref, a_ref, b_ref, pb_ref, hr_ref, hc_ref):
    h = h_ref[...]
    pb = pb_ref[...]
    mr = jnp.dot(h, a_ref[...],
                 preferred_element_type=jnp.float32).astype(jnp.bfloat16)
    mc = jnp.dot(h, b_ref[...],
                 preferred_element_type=jnp.float32).astype(jnp.bfloat16)
    hr_ref[...] = jnp.concatenate([mr, pb], axis=1)
    hc_ref[...] = jnp.concatenate([mc, pb], axis=1)


def _prep(h, wa, wb, posbf, bn):
    n, d = h.shape
    dt = d + posbf.shape[1]
    grid = n // bn
    return pl.pallas_call(
        _prep_body,
        grid=(grid,),
        in_specs=[
            pl.BlockSpec((bn, d), lambda i: (i, 0)),
            pl.BlockSpec((d, d), lambda i: (0, 0)),
            pl.BlockSpec((d, d), lambda i: (0, 0)),
            pl.BlockSpec((bn, posbf.shape[1]), lambda i: (i, 0)),
        ],
        out_specs=[
            pl.BlockSpec((bn, dt), lambda i: (i, 0)),
            pl.BlockSpec((bn, dt), lambda i: (i, 0)),
        ],
        out_shape=[
            jax.ShapeDtypeStruct((n, dt), jnp.bfloat16),
            jax.ShapeDtypeStruct((n, dt), jnp.bfloat16),
        ],
        compiler_params=pltpu.CompilerParams(
            dimension_semantics=("parallel",)),
    )(h, wa, wb, posbf)


def _gather_body(off, mch, rc_hbm, hr_hbm, hc_hbm,
                 g_hbm, dif_hbm,
                 idx, buf_a, buf_b, gb, db, sem0, sem1, wsem0, wsem1):
    d = 128
    wid = lax.axis_index("s") * NC + lax.axis_index("c")
    w_base = wid * mch * CH
    sems = (sem0, sem1)
    wsems = (wsem0, wsem1)
    comp = plsc.PackFormat.INTERLEAVED

    def fire(c, b):
        base = w_base + c * CH
        pltpu.sync_copy(rc_hbm.at[pl.ds((off + base) * 2, 2 * CH)], idx.at[b])
        pltpu.async_copy(hr_hbm.at[idx.at[b, pl.ds(0, CH)]], buf_a.at[b],
                         sems[b])
        pltpu.async_copy(hc_hbm.at[idx.at[b, pl.ds(CH, CH)]], buf_b.at[b],
                         sems[b])

    def wait(b):
        pltpu.make_async_copy(hr_hbm.at[idx.at[b, pl.ds(0, CH)]],
                              buf_a.at[b], sems[b]).wait()
        pltpu.make_async_copy(hc_hbm.at[idx.at[b, pl.ds(CH, CH)]],
                              buf_b.at[b], sems[b]).wait()

    def fire_writes(c, b):
        base = w_base + c * CH
        pltpu.async_copy(gb.at[b], g_hbm.at[pl.ds(base, CH)], wsems[b])
        pltpu.async_copy(db.at[b], dif_hbm.at[pl.ds(base, CH)], wsems[b])

    def wait_writes(c, b):
        base = w_base + c * CH
        pltpu.make_async_copy(gb.at[b], g_hbm.at[pl.ds(base, CH)],
                              wsems[b]).wait()
        pltpu.make_async_copy(db.at[b], dif_hbm.at[pl.ds(base, CH)],
                              wsems[b]).wait()

    fire(0, 0)

    @pl.loop(0, mch // 2)
    def _pair(i):
        for b in range(2):
            j = i * 2 + b

            @pl.when(j < mch - 1)
            def _():
                @pl.when(j >= 1)
                def _():
                    wait_writes(j - 1, 1 - b)

                fire(j + 1, 1 - b)

            wait(b)

            @pl.loop(0, CH, unroll=2)
            def _row(r):
                for c in range(d // 32):
                    sl = pl.ds(c * 32, 32)
                    a0, a1 = plsc.unpack(
                        buf_a[b, r, sl], format=comp,
                        preferred_element_type=jnp.float32)
                    b0, b1 = plsc.unpack(
                        buf_b[b, r, sl], format=comp,
                        preferred_element_type=jnp.float32)
                    gb[b, r, pl.ds(c * 32, 16)] = a0 + b0
                    gb[b, r, pl.ds(c * 32 + 16, 16)] = a1 + b1
                pr = plsc.bitcast(buf_a[b, r, pl.ds(d, 32)], jnp.float32)
                pc = plsc.bitcast(buf_b[b, r, pl.ds(d, 32)], jnp.float32)
                db[b, r, :] = pr - pc

            fire_writes(j, b)

    wait_writes(mch - 2, 0)
    wait_writes(mch - 1, 1)


def _gather(rc, hrp, hcp, off, mch):
    d = 128
    dt = hrp.shape[1]
    epl = NW * CH * mch
    mesh = plsc.VectorSubcoreMesh(core_axis_name="c", subcore_axis_name="s",
                                  num_cores=NC, num_subcores=NS)
    f = pl.kernel(
        functools.partial(_gather_body, off, mch),
        out_type=[
            jax.ShapeDtypeStruct((epl, d), jnp.float32),
            jax.ShapeDtypeStruct((epl, PP), jnp.float32),
        ],
        mesh=mesh,
        scratch_types=[
            pltpu.VMEM((2, 2 * CH), jnp.int32),
            pltpu.VMEM((2, CH, dt), jnp.bfloat16),
            pltpu.VMEM((2, CH, dt), jnp.bfloat16),
            pltpu.VMEM((2, CH, d), jnp.float32),
            pltpu.VMEM((2, CH, PP), jnp.float32),
            pltpu.SemaphoreType.DMA,
            pltpu.SemaphoreType.DMA,
            pltpu.SemaphoreType.DMA,
            pltpu.SemaphoreType.DMA,
        ],
        compiler_params=pltpu.CompilerParams(use_tc_tiling_on_sc=False,
                                             needs_layout_passes=False),
    )
    return f(rc, hrp, hcp)


def _edge_body(ne, be, off, g_ref, eat_ref, dif_ref,
               wea_ref, wrad_ref, be1_ref, we2_ref, be2_ref,
               wc1_ref, bc1_ref, wc2_ref,
               e_ref, t_ref):
    bf = jnp.bfloat16
    diff = dif_ref[...]
    radial = jnp.sqrt(jnp.sum(diff * diff, axis=1, keepdims=True)) + EPS
    pre = (g_ref[...]
           + lax.dot_general(eat_ref[...].astype(bf),
                             wea_ref[...].astype(bf),
                             dimension_numbers=(((0,), (0,)), ((), ())),
                             preferred_element_type=jnp.float32)
           + radial * wrad_ref[...]
           + be1_ref[...])
    e1 = _silu(pre)
    e2 = _silu(jnp.dot(e1.astype(bf), we2_ref[...].astype(bf),
                       preferred_element_type=jnp.float32) + be2_ref[...])
    c1 = _silu(jnp.dot(e2.astype(bf), wc1_ref[...].astype(bf),
                       preferred_element_type=jnp.float32) + bc1_ref[...])
    cu = jnp.sum(c1 * wc2_ref[...], axis=1, keepdims=True)
    cu = jnp.clip(cu, -1.0, 1.0)
    erow = (off + pl.program_id(0) * be
            + lax.broadcasted_iota(jnp.int32, (be, 1), 0))
    valid = erow < ne
    e_ref[...] = jnp.where(valid, e2, 0.0)
    t_ref[...] = jnp.where(valid, cu * diff, 0.0)


def _edge_mlp(g, eat, dif, wea, wrad, be1, we2, be2, wc1, bc1, wc2,
              ne, be, off):
    ep, d = g.shape
    ed = eat.shape[0]
    grid = ep // be
    ea_max = ne // be - 1
    ea_off = off // be
    full = lambda i: (0, 0)
    return pl.pallas_call(
        functools.partial(_edge_body, ne, be, off),
        grid=(grid,),
        in_specs=[
            pl.BlockSpec((be, d), lambda i: (i, 0)),
            pl.BlockSpec((ed, be),
                         lambda i, m=ea_max, o=ea_off:
                         (0, jnp.minimum(i + o, m))),
            pl.BlockSpec((be, PP), lambda i: (i, 0)),
            pl.BlockSpec((ed, d), full),
            pl.BlockSpec((1, d), full),
            pl.BlockSpec((1, d), full),
            pl.BlockSpec((d, d), full),
            pl.BlockSpec((1, d), full),
            pl.BlockSpec((d, d), full),
            pl.BlockSpec((1, d), full),
            pl.BlockSpec((1, d), full),
        ],
        out_specs=[
            pl.BlockSpec((be, d), lambda i: (i, 0)),
            pl.BlockSpec((be, PP), lambda i: (i, 0)),
        ],
        out_shape=[
            jax.ShapeDtypeStruct((ep, d), jnp.float32),
            jax.ShapeDtypeStruct((ep, PP), jnp.float32),
        ],
        compiler_params=pltpu.CompilerParams(
            dimension_semantics=("parallel",)),
    )(g, eat, dif, wea, wrad, be1, we2, be2, wc1, bc1, wc2)


def _scatter_body(off, mch, row_hbm, e_hbm, t_hbm, zn_hbm, zc_hbm,
                  outn_hbm, outc_hbm,
                  idx, ebuf, tbuf, accn, accc, sem0, sem1):
    n = zn_hbm.shape[0]
    rows_per_s = n // NS
    c = lax.axis_index("c")
    s = lax.axis_index("s")
    wid = s * NC + c
    w_base = wid * mch * CH
    sems = (sem0, sem1)

    pltpu.sync_copy(zn_hbm.at[pl.ds(s * rows_per_s, rows_per_s)],
                    accn.at[pl.ds(s * rows_per_s, rows_per_s)])
    pltpu.sync_copy(zc_hbm.at[pl.ds(s * rows_per_s, rows_per_s)],
                    accc.at[pl.ds(s * rows_per_s, rows_per_s)])
    plsc.subcore_barrier()

    def fire(j, b):
        base = w_base + j * CH
        pltpu.async_copy(row_hbm.at[pl.ds(off + base, CH)], idx.at[b],
                         sems[b])
        pltpu.async_copy(e_hbm.at[pl.ds(base, CH)], ebuf.at[b], sems[b])
        pltpu.async_copy(t_hbm.at[pl.ds(base, CH)], tbuf.at[b], sems[b])

    def wait(j, b):
        base = w_base + j * CH
        pltpu.make_async_copy(row_hbm.at[pl.ds(off + base, CH)], idx.at[b],
                              sems[b]).wait()
        pltpu.make_async_copy(e_hbm.at[pl.ds(base, CH)], ebuf.at[b],
                              sems[b]).wait()
        pltpu.make_async_copy(t_hbm.at[pl.ds(base, CH)], tbuf.at[b],
                              sems[b]).wait()

    fire(0, 0)

    @pl.loop(0, mch // 2)
    def _pair(i):
        for b in range(2):
            j = i * 2 + b

            @pl.when(j < mch - 1)
            def _():
                fire(j + 1, 1 - b)

            wait(j, b)
            pltpu.sync_copy(ebuf.at[b], accn.at[idx.at[b]], add=True)
            pltpu.sync_copy(tbuf.at[b], accc.at[idx.at[b]], add=True)

    plsc.subcore_barrier()
    pltpu.sync_copy(accn.at[pl.ds(s * rows_per_s, rows_per_s)],
                    outn_hbm.at[pl.ds(c * n + s * rows_per_s, rows_per_s)])
    pltpu.sync_copy(accc.at[pl.ds(s * rows_per_s, rows_per_s)],
                    outc_hbm.at[pl.ds(c * n + s * rows_per_s, rows_per_s)])


def _scatter(row, earr, tarr, n, off, mch):
    ep, d = earr.shape
    zn = jnp.zeros((n, d), jnp.float32)
    zc = jnp.zeros((n, PP), jnp.float32)
    mesh = plsc.VectorSubcoreMesh(core_axis_name="c", subcore_axis_name="s",
                                  num_cores=NC, num_subcores=NS)
    f = pl.kernel(
        functools.partial(_scatter_body, off, mch),
        out_type=[
            jax.ShapeDtypeStruct((NC * n, d), jnp.float32),
            jax.ShapeDtypeStruct((NC * n, PP), jnp.float32),
        ],
        mesh=mesh,
        scratch_types=[
            pltpu.VMEM((2, CH), jnp.int32),
            pltpu.VMEM((2, CH, d), jnp.float32),
            pltpu.VMEM((2, CH, PP), jnp.float32),
            pltpu.VMEM_SHARED((n, d), jnp.float32),
            pltpu.VMEM_SHARED((n, PP), jnp.float32),
            pltpu.SemaphoreType.DMA,
            pltpu.SemaphoreType.DMA,
        ],
        compiler_params=pltpu.CompilerParams(use_tc_tiling_on_sc=False),
    )
    return f(row, earr, tarr, zn, zc)


def _node_body(h_ref, n1_ref, n2_ref, n3_ref, n4_ref,
               c1_ref, c2_ref, c3_ref, c4_ref, pp_ref,
               wn1a_ref, wn1b_ref, bn1_ref, wn2_ref, bn2_ref,
               hn_ref, pn_ref):
    h = h_ref[...]
    an = n1_ref[...] + n2_ref[...] + n3_ref[...] + n4_ref[...]
    x = _silu(jnp.dot(h, wn1a_ref[...], preferred_element_type=jnp.float32)
              + jnp.dot(an, wn1b_ref[...], preferred_element_type=jnp.float32)
              + bn1_ref[...])
    hn_ref[...] = (jnp.dot(x, wn2_ref[...], preferred_element_type=jnp.float32)
                   + bn2_ref[...] + h)
    pn_ref[...] = (pp_ref[...] + c1_ref[...] + c2_ref[...]
                   + c3_ref[...] + c4_ref[...])


def _node_mlp(h, outn_a, outc_a, outn_b, outc_b, pos_pad,
              wn1a, wn1b, bn1, wn2, bn2, bn):
    n, d = h.shape
    grid = n // bn
    full = lambda i: (0, 0)
    return pl.pallas_call(
        _node_body,
        grid=(grid,),
        in_specs=[
            pl.BlockSpec((bn, d), lambda i: (i, 0)),
            pl.BlockSpec((bn, d), lambda i: (i, 0)),
            pl.BlockSpec((bn, d), lambda i, g=grid: (i + g, 0)),
            pl.BlockSpec((bn, d), lambda i: (i, 0)),
            pl.BlockSpec((bn, d), lambda i, g=grid: (i + g, 0)),
            pl.BlockSpec((bn, PP), lambda i: (i, 0)),
            pl.BlockSpec((bn, PP), lambda i, g=grid: (i + g, 0)),
            pl.BlockSpec((bn, PP), lambda i: (i, 0)),
            pl.BlockSpec((bn, PP), lambda i, g=grid: (i + g, 0)),
            pl.BlockSpec((bn, PP), lambda i: (i, 0)),
            pl.BlockSpec((d, d), full),
            pl.BlockSpec((d, d), full),
            pl.BlockSpec((1, d), full),
            pl.BlockSpec((d, d), full),
            pl.BlockSpec((1, d), full),
        ],
        out_specs=[
            pl.BlockSpec((bn, d), lambda i: (i, 0)),
            pl.BlockSpec((bn, PP), lambda i: (i, 0)),
        ],
        out_shape=[
            jax.ShapeDtypeStruct((n, d), jnp.float32),
            jax.ShapeDtypeStruct((n, PP), jnp.float32),
        ],
        compiler_params=pltpu.CompilerParams(
            dimension_semantics=("parallel",)),
    )(h, outn_a, outn_a, outn_b, outn_b, outc_a, outc_a, outc_b, outc_b,
      pos_pad, wn1a, wn1b, bn1, wn2, bn2)


def kernel(h, edge_index, edge_attr, pos, We1, be1, We2, be2,
           Wc1, bc1, Wc2, Wn1, bn1, Wn2, bn2):
    n, d = h.shape
    e = edge_index.shape[1]
    ed = edge_attr.shape[1]

    row = jnp.zeros((EP,), jnp.int32).at[:e].set(edge_index[0])
    col = jnp.zeros((EP,), jnp.int32).at[:e].set(edge_index[1])
    pos_pad = jnp.zeros((n, PP), jnp.float32).at[:, :3].set(pos)
    posbf = lax.bitcast_convert_type(pos_pad, jnp.bfloat16).reshape(n, 2 * PP)
    rc = jnp.stack([row.reshape(-1, CH), col.reshape(-1, CH)],
                   axis=1).reshape(-1)

    wa = We1[:d][:, _PERM]
    wb = We1[d:2 * d][:, _PERM]
    wea = We1[2 * d:2 * d + ed]
    wrad = We1[2 * d + ed:]

    hrp, hcp = _prep(h, wa, wb, posbf, bn=2000)

    eat = edge_attr.T
    mch_p = MCH // 2
    half = EP // 2
    parts = []
    for p in range(2):
        off = p * half
        gp, difp = _gather(rc, hrp, hcp, off=off, mch=mch_p)
        ep_, tp = _edge_mlp(gp, eat, difp,
                            wea, wrad, be1.reshape(1, -1), We2,
                            be2.reshape(1, -1), Wc1, bc1.reshape(1, -1),
                            Wc2.reshape(1, -1), ne=e, be=2560, off=off)
        parts.append(_scatter(row, ep_, tp, n, off=off, mch=mch_p))

    (outn_a, outc_a), (outn_b, outc_b) = parts
    h_new, pn = _node_mlp(h, outn_a, outc_a, outn_b, outc_b, pos_pad,
                          Wn1[:d], Wn1[d:], bn1.reshape(1, -1),
                          Wn2, bn2.reshape(1, -1), bn=2000)

    return (h_new, pn[:, :3])


# speedup vs baseline: 1.7771x; 1.1812x over previous
import functools

import numpy as np
import jax
import jax.numpy as jnp
from jax import lax
from jax.experimental import pallas as pl
from jax.experimental.pallas import tpu as pltpu
from jax.experimental.pallas import tpu_sc as plsc

EPS = 1e-08

NC = 2
NS = 16
NW = NC * NS
CH = 128
MCH = 80
EP = NW * CH * MCH
PP = 16


def _silu(x):
    return x * jax.nn.sigmoid(x)


_PERM = np.zeros(128, np.int32)
for _c in range(4):
    for _i in range(16):
        _PERM[_c * 32 + 2 * _i] = _c * 32 + _i
        _PERM[_c * 32 + 2 * _i + 1] = _c * 32 + 16 + _i


def _prep_body(h_ref, a_ref, b_ref, pb_ref, hr_ref, hc_ref):
    h = h_ref[...]
    pb = pb_ref[...]
    mr = jnp.dot(h, a_ref[...],
                 preferred_element_type=jnp.float32).astype(jnp.bfloat16)
    mc = jnp.dot(h, b_ref[...],
                 preferred_element_type=jnp.float32).astype(jnp.bfloat16)
    hr_ref[...] = jnp.concatenate([mr, pb], axis=1)
    hc_ref[...] = jnp.concatenate([mc, pb], axis=1)


def _prep(h, wa, wb, posbf, bn):
    n, d = h.shape
    dt = d + posbf.shape[1]
    grid = n // bn
    return pl.pallas_call(
        _prep_body,
        grid=(grid,),
        in_specs=[
            pl.BlockSpec((bn, d), lambda i: (i, 0)),
            pl.BlockSpec((d, d), lambda i: (0, 0)),
            pl.BlockSpec((d, d), lambda i: (0, 0)),
            pl.BlockSpec((bn, posbf.shape[1]), lambda i: (i, 0)),
        ],
        out_specs=[
            pl.BlockSpec((bn, dt), lambda i: (i, 0)),
            pl.BlockSpec((bn, dt), lambda i: (i, 0)),
        ],
        out_shape=[
            jax.ShapeDtypeStruct((n, dt), jnp.bfloat16),
            jax.ShapeDtypeStruct((n, dt), jnp.bfloat16),
        ],
        compiler_params=pltpu.CompilerParams(
            dimension_semantics=("parallel",)),
    )(h, wa, wb, posbf)


def _gather_body(off, mch, rc_hbm, hr_hbm, hc_hbm,
                 g_hbm, dif_hbm,
                 idx, buf_a, buf_b, gb, db, sem0, sem1, wsem0, wsem1):
    d = 128
    wid = lax.axis_index("s") * NC + lax.axis_index("c")
    w_base = wid * mch * CH
    sems = (sem0, sem1)
    wsems = (wsem0, wsem1)
    comp = plsc.PackFormat.INTERLEAVED

    def fire(c, b):
        base = w_base + c * CH
        pltpu.sync_copy(rc_hbm.at[pl.ds((off + base) * 2, 2 * CH)], idx.at[b])
        pltpu.async_copy(hr_hbm.at[idx.at[b, pl.ds(0, CH)]], buf_a.at[b],
                         sems[b])
        pltpu.async_copy(hc_hbm.at[idx.at[b, pl.ds(CH, CH)]], buf_b.at[b],
                         sems[b])

    def wait(b):
        pltpu.make_async_copy(hr_hbm.at[idx.at[b, pl.ds(0, CH)]],
                              buf_a.at[b], sems[b]).wait()
        pltpu.make_async_copy(hc_hbm.at[idx.at[b, pl.ds(CH, CH)]],
                              buf_b.at[b], sems[b]).wait()

    def _planar(c):
        g = wid * mch + c
        row0 = (g // 16) * 256 + (g % 2) * 128
        mlane = ((g // 2) % 8) * 16
        return row0, mlane

    def fire_writes(c, b):
        base = w_base + c * CH
        row0, mlane = _planar(c)
        pltpu.async_copy(gb.at[b], g_hbm.at[pl.ds(base, CH)], wsems[b])
        pltpu.async_copy(db.at[b],
                         dif_hbm.at[pl.ds(row0, CH), pl.ds(mlane, PP)],
                         wsems[b])

    def wait_writes(c, b):
        base = w_base + c * CH
        row0, mlane = _planar(c)
        pltpu.make_async_copy(gb.at[b], g_hbm.at[pl.ds(base, CH)],
                              wsems[b]).wait()
        pltpu.make_async_copy(db.at[b],
                              dif_hbm.at[pl.ds(row0, CH), pl.ds(mlane, PP)],
                              wsems[b]).wait()

    fire(0, 0)

    @pl.loop(0, mch // 2)
    def _pair(i):
        for b in range(2):
            j = i * 2 + b

            @pl.when(j < mch - 1)
            def _():
                @pl.when(j >= 1)
                def _():
                    wait_writes(j - 1, 1 - b)

                fire(j + 1, 1 - b)

            wait(b)

            @pl.loop(0, CH, unroll=2)
            def _row(r):
                for c in range(d // 32):
                    sl = pl.ds(c * 32, 32)
                    a0, a1 = plsc.unpack(
                        buf_a[b, r, sl], format=comp,
                        preferred_element_type=jnp.float32)
                    b0, b1 = plsc.unpack(
                        buf_b[b, r, sl], format=comp,
                        preferred_element_type=jnp.float32)
                    gb[b, r, pl.ds(c * 32, 16)] = a0 + b0
                    gb[b, r, pl.ds(c * 32 + 16, 16)] = a1 + b1
                pr = plsc.bitcast(buf_a[b, r, pl.ds(d, 32)], jnp.float32)
                pc = plsc.bitcast(buf_b[b, r, pl.ds(d, 32)], jnp.float32)
                db[b, r, :] = pr - pc

            fire_writes(j, b)

    wait_writes(mch - 2, 0)
    wait_writes(mch - 1, 1)


def _gather(rc, hrp, hcp, off, mch):
    d = 128
    dt = hrp.shape[1]
    epl = NW * CH * mch
    mesh = plsc.VectorSubcoreMesh(core_axis_name="c", subcore_axis_name="s",
                                  num_cores=NC, num_subcores=NS)
    f = pl.kernel(
        functools.partial(_gather_body, off, mch),
        out_type=[
            jax.ShapeDtypeStruct((epl, d), jnp.float32),
            jax.ShapeDtypeStruct((epl // 8, 128), jnp.float32),
        ],
        mesh=mesh,
        scratch_types=[
            pltpu.VMEM((2, 2 * CH), jnp.int32),
            pltpu.VMEM((2, CH, dt), jnp.bfloat16),
            pltpu.VMEM((2, CH, dt), jnp.bfloat16),
            pltpu.VMEM((2, CH, d), jnp.float32),
            pltpu.VMEM((2, CH, PP), jnp.float32),
            pltpu.SemaphoreType.DMA,
            pltpu.SemaphoreType.DMA,
            pltpu.SemaphoreType.DMA,
            pltpu.SemaphoreType.DMA,
        ],
        compiler_params=pltpu.CompilerParams(use_tc_tiling_on_sc=False,
                                             needs_layout_passes=False),
    )
    return f(rc, hrp, hcp)


def _edge_body(ne, be, off, g_ref, eat_ref, dif_ref,
               wea_ref, wrad_ref, be1_ref, we2_ref, be2_ref,
               wc1_ref, bc1_ref, wc2_ref,
               e_ref, t_ref):
    bf = jnp.bfloat16
    d8 = dif_ref[...]
    diff = jnp.concatenate([d8[:, 16 * m:16 * (m + 1)] for m in range(8)],
                           axis=0)
    radial = jnp.sqrt(jnp.sum(diff * diff, axis=1, keepdims=True)) + EPS
    pre = (g_ref[...]
           + lax.dot_general(eat_ref[...].astype(bf),
                             wea_ref[...].astype(bf),
                             dimension_numbers=(((0,), (0,)), ((), ())),
                             preferred_element_type=jnp.float32)
           + radial * wrad_ref[...]
           + be1_ref[...])
    e1 = _silu(pre)
    e2 = _silu(jnp.dot(e1.astype(bf), we2_ref[...].astype(bf),
                       preferred_element_type=jnp.float32) + be2_ref[...])
    c1 = _silu(jnp.dot(e2.astype(bf), wc1_ref[...].astype(bf),
                       preferred_element_type=jnp.float32) + bc1_ref[...])
    cu = jnp.sum(c1 * wc2_ref[...], axis=1, keepdims=True)
    cu = jnp.clip(cu, -1.0, 1.0)
    erow = (off + pl.program_id(0) * be
            + lax.broadcasted_iota(jnp.int32, (be, 1), 0))
    valid = erow < ne
    e_ref[...] = jnp.where(valid, e2, 0.0)
    t16 = jnp.where(valid, cu * diff, 0.0)
    t_ref[...] = jnp.concatenate(
        [t16[(be // 8) * m:(be // 8) * (m + 1), :] for m in range(8)], axis=1)


def _edge_mlp(g, eat, dif, wea, wrad, be1, we2, be2, wc1, bc1, wc2,
              ne, be, off):
    ep, d = g.shape
    ed = eat.shape[0]
    grid = ep // be
    ea_off = off // be
    full = lambda i: (0, 0)
    return pl.pallas_call(
        functools.partial(_edge_body, ne, be, off),
        grid=(grid,),
        in_specs=[
            pl.BlockSpec((be, d), lambda i: (i, 0)),
            pl.BlockSpec((ed, be), lambda i, o=ea_off: (0, i + o)),
            pl.BlockSpec((be // 8, 128), lambda i: (i, 0)),
            pl.BlockSpec((ed, d), full),
            pl.BlockSpec((1, d), full),
            pl.BlockSpec((1, d), full),
            pl.BlockSpec((d, d), full),
            pl.BlockSpec((1, d), full),
            pl.BlockSpec((d, d), full),
            pl.BlockSpec((1, d), full),
            pl.BlockSpec((1, d), full),
        ],
        out_specs=[
            pl.BlockSpec((be, d), lambda i: (i, 0)),
            pl.BlockSpec((be // 8, 128), lambda i: (i, 0)),
        ],
        out_shape=[
            jax.ShapeDtypeStruct((ep, d), jnp.float32),
            jax.ShapeDtypeStruct((ep // 8, 128), jnp.float32),
        ],
        compiler_params=pltpu.CompilerParams(
            dimension_semantics=("parallel",)),
    )(g, eat, dif, wea, wrad, be1, we2, be2, wc1, bc1, wc2)


def _scatter_body(off, mch, row_hbm, e_hbm, t_hbm, zn_hbm, zc_hbm,
                  outn_hbm, outc_hbm,
                  idx, ebuf, tbuf, accn, accc, sem0, sem1):
    n = zn_hbm.shape[0]
    rows_per_s = n // NS
    c = lax.axis_index("c")
    s = lax.axis_index("s")
    wid = s * NC + c
    w_base = wid * mch * CH
    sems = (sem0, sem1)

    pltpu.sync_copy(zn_hbm.at[pl.ds(s * rows_per_s, rows_per_s)],
                    accn.at[pl.ds(s * rows_per_s, rows_per_s)])
    pltpu.sync_copy(zc_hbm.at[pl.ds(s * rows_per_s, rows_per_s)],
                    accc.at[pl.ds(s * rows_per_s, rows_per_s)])
    plsc.subcore_barrier()

    def _planar(j):
        g = wid * mch + j
        row0 = (g // 16) * 256 + (g % 2) * 128
        mlane = ((g // 2) % 8) * 16
        return row0, mlane

    def fire(j, b):
        base = w_base + j * CH
        row0, mlane = _planar(j)
        pltpu.async_copy(row_hbm.at[pl.ds(off + base, CH)], idx.at[b],
                         sems[b])
        pltpu.async_copy(e_hbm.at[pl.ds(base, CH)], ebuf.at[b], sems[b])
        pltpu.async_copy(t_hbm.at[pl.ds(row0, CH), pl.ds(mlane, PP)],
                         tbuf.at[b], sems[b])

    def wait(j, b):
        base = w_base + j * CH
        row0, mlane = _planar(j)
        pltpu.make_async_copy(row_hbm.at[pl.ds(off + base, CH)], idx.at[b],
                              sems[b]).wait()
        pltpu.make_async_copy(e_hbm.at[pl.ds(base, CH)], ebuf.at[b],
                              sems[b]).wait()
        pltpu.make_async_copy(t_hbm.at[pl.ds(row0, CH), pl.ds(mlane, PP)],
                              tbuf.at[b], sems[b]).wait()

    fire(0, 0)

    @pl.loop(0, mch // 2)
    def _pair(i):
        for b in range(2):
            j = i * 2 + b

            @pl.when(j < mch - 1)
            def _():
                fire(j + 1, 1 - b)

            wait(j, b)
            pltpu.sync_copy(ebuf.at[b], accn.at[idx.at[b]], add=True)
            pltpu.sync_copy(tbuf.at[b], accc.at[idx.at[b]], add=True)

    plsc.subcore_barrier()
    pltpu.sync_copy(accn.at[pl.ds(s * rows_per_s, rows_per_s)],
                    outn_hbm.at[pl.ds(c * n + s * rows_per_s, rows_per_s)])
    pltpu.sync_copy(accc.at[pl.ds(s * rows_per_s, rows_per_s)],
                    outc_hbm.at[pl.ds(c * n + s * rows_per_s, rows_per_s)])


def _scatter(row, earr, tarr, n, off, mch):
    ep, d = earr.shape
    zn = jnp.zeros((n, d), jnp.float32)
    zc = jnp.zeros((n, PP), jnp.float32)
    mesh = plsc.VectorSubcoreMesh(core_axis_name="c", subcore_axis_name="s",
                                  num_cores=NC, num_subcores=NS)
    f = pl.kernel(
        functools.partial(_scatter_body, off, mch),
        out_type=[
            jax.ShapeDtypeStruct((NC * n, d), jnp.float32),
            jax.ShapeDtypeStruct((NC * n, PP), jnp.float32),
        ],
        mesh=mesh,
        scratch_types=[
            pltpu.VMEM((2, CH), jnp.int32),
            pltpu.VMEM((2, CH, d), jnp.float32),
            pltpu.VMEM((2, CH, PP), jnp.float32),
            pltpu.VMEM_SHARED((n, d), jnp.float32),
            pltpu.VMEM_SHARED((n, PP), jnp.float32),
            pltpu.SemaphoreType.DMA,
            pltpu.SemaphoreType.DMA,
        ],
        compiler_params=pltpu.CompilerParams(use_tc_tiling_on_sc=False),
    )
    return f(row, earr, tarr, zn, zc)


def _node_body(h_ref, n1_ref, n2_ref, n3_ref, n4_ref,
               c1_ref, c2_ref, c3_ref, c4_ref, pp_ref,
               wn1a_ref, wn1b_ref, bn1_ref, wn2_ref, bn2_ref,
               hn_ref, pn_ref):
    h = h_ref[...]
    an = n1_ref[...] + n2_ref[...] + n3_ref[...] + n4_ref[...]
    x = _silu(jnp.dot(h, wn1a_ref[...], preferred_element_type=jnp.float32)
              + jnp.dot(an, wn1b_ref[...], preferred_element_type=jnp.float32)
              + bn1_ref[...])
    hn_ref[...] = (jnp.dot(x, wn2_ref[...], preferred_element_type=jnp.float32)
                   + bn2_ref[...] + h)
    pn_ref[...] = (pp_ref[...] + c1_ref[...] + c2_ref[...]
                   + c3_ref[...] + c4_ref[...])


def _node_mlp(h, outn_a, outc_a, outn_b, outc_b, pos_pad,
              wn1a, wn1b, bn1, wn2, bn2, bn):
    n, d = h.shape
    grid = n // bn
    full = lambda i: (0, 0)
    return pl.pallas_call(
        _node_body,
        grid=(grid,),
        in_specs=[
            pl.BlockSpec((bn, d), lambda i: (i, 0)),
            pl.BlockSpec((bn, d), lambda i: (i, 0)),
            pl.BlockSpec((bn, d), lambda i, g=grid: (i + g, 0)),
            pl.BlockSpec((bn, d), lambda i: (i, 0)),
            pl.BlockSpec((bn, d), lambda i, g=grid: (i + g, 0)),
            pl.BlockSpec((bn, PP), lambda i: (i, 0)),
            pl.BlockSpec((bn, PP), lambda i, g=grid: (i + g, 0)),
            pl.BlockSpec((bn, PP), lambda i: (i, 0)),
            pl.BlockSpec((bn, PP), lambda i, g=grid: (i + g, 0)),
            pl.BlockSpec((bn, PP), lambda i: (i, 0)),
            pl.BlockSpec((d, d), full),
            pl.BlockSpec((d, d), full),
            pl.BlockSpec((1, d), full),
            pl.BlockSpec((d, d), full),
            pl.BlockSpec((1, d), full),
        ],
        out_specs=[
            pl.BlockSpec((bn, d), lambda i: (i, 0)),
            pl.BlockSpec((bn, PP), lambda i: (i, 0)),
        ],
        out_shape=[
            jax.ShapeDtypeStruct((n, d), jnp.float32),
            jax.ShapeDtypeStruct((n, PP), jnp.float32),
        ],
        compiler_params=pltpu.CompilerParams(
            dimension_semantics=("parallel",)),
    )(h, outn_a, outn_a, outn_b, outn_b, outc_a, outc_a, outc_b, outc_b,
      pos_pad, wn1a, wn1b, bn1, wn2, bn2)


def kernel(h, edge_index, edge_attr, pos, We1, be1, We2, be2,
           Wc1, bc1, Wc2, Wn1, bn1, Wn2, bn2):
    n, d = h.shape
    e = edge_index.shape[1]
    ed = edge_attr.shape[1]

    row = jnp.zeros((EP,), jnp.int32).at[:e].set(edge_index[0])
    col = jnp.zeros((EP,), jnp.int32).at[:e].set(edge_index[1])
    pos_pad = jnp.zeros((n, PP), jnp.float32).at[:, :3].set(pos)
    posbf = lax.bitcast_convert_type(pos_pad, jnp.bfloat16).reshape(n, 2 * PP)
    rc = jnp.stack([row.reshape(-1, CH), col.reshape(-1, CH)],
                   axis=1).reshape(-1)

    wa = We1[:d][:, _PERM]
    wb = We1[d:2 * d][:, _PERM]
    wea = We1[2 * d:2 * d + ed]
    wrad = We1[2 * d + ed:]

    hrp, hcp = _prep(h, wa, wb, posbf, bn=2000)

    eat = jnp.zeros((ed, EP), jnp.float32).at[:, :e].set(edge_attr.T)
    mch_p = MCH // 2
    half = EP // 2
    parts = []
    for p in range(2):
        off = p * half
        gp, difp = _gather(rc, hrp, hcp, off=off, mch=mch_p)
        ep_, tp = _edge_mlp(gp, eat, difp,
                            wea, wrad, be1.reshape(1, -1), We2,
                            be2.reshape(1, -1), Wc1, bc1.reshape(1, -1),
                            Wc2.reshape(1, -1), ne=e, be=2048, off=off)
        parts.append(_scatter(row, ep_, tp, n, off=off, mch=mch_p))

    (outn_a, outc_a), (outn_b, outc_b) = parts
    h_new, pn = _node_mlp(h, outn_a, outc_a, outn_b, outc_b, pos_pad,
                          Wn1[:d], Wn1[d:], bn1.reshape(1, -1),
                          Wn2, bn2.reshape(1, -1), bn=2000)

    return (h_new, pn[:, :3])
